# Initial kernel scaffold; baseline (speedup 1.0000x reference)
import jax
import jax.numpy as jnp
from jax.experimental import pallas as pl


def kernel(edge_index, edge_weight, num_nodes, W_e1, b_e1, W_e2, b_e2, W_n1, b_n1, W_n2, b_n2):
    raise NotImplementedError("write your pallas kernel here")


# speedup vs baseline: 23.8532x; 23.8532x over previous
import functools

import jax
import jax.numpy as jnp
from jax import lax
from jax.experimental import pallas as pl
from jax.experimental.pallas import tpu as pltpu
from jax.experimental.pallas import tpu_sc as plsc

N_NODES = 100000
N_EDGES = 1600000
NDIM = 32

NUM_CORES = 2
NUM_SUBCORES = 16
NW = NUM_CORES * NUM_SUBCORES
E_PER_W = N_EDGES // NW
CHUNK = 2000
N_CHUNKS = E_PER_W // CHUNK
GROUPS = CHUNK // 16
LANES = 16


def _sc_histogram(edge_flat, off16):
    mesh = plsc.VectorSubcoreMesh(
        core_axis_name="c", subcore_axis_name="s",
        num_cores=NUM_CORES, num_subcores=NUM_SUBCORES)

    @functools.partial(
        pl.kernel,
        out_type=jax.ShapeDtypeStruct((NW, N_NODES), jnp.float32),
        mesh=mesh,
        scratch_types=[
            pltpu.VMEM((N_NODES,), jnp.float32),
            pltpu.VMEM((CHUNK,), jnp.int32),
            pltpu.VMEM((CHUNK,), jnp.int32),
            pltpu.VMEM((LANES,), jnp.int32),
            pltpu.SemaphoreType.DMA,
            pltpu.SemaphoreType.DMA,
        ],
        compiler_params=pltpu.CompilerParams(needs_layout_passes=False),
    )
    def hist_kernel(edge_hbm, off_hbm, out_hbm, hist, idxbuf0, idxbuf1,
                    off_v, sem0, sem1):
        wid = lax.axis_index("s") * NUM_CORES + lax.axis_index("c")
        e_base = wid * E_PER_W
        sems = (sem0, sem1)
        bufs = (idxbuf0, idxbuf1)

        pltpu.sync_copy(off_hbm, off_v)
        off = off_v[...]

        @pl.loop(0, N_NODES // LANES, unroll=8)
        def _zero(i):
            hist[pl.ds(i * LANES, LANES)] = jnp.zeros((LANES,), jnp.float32)

        ones = jnp.ones((LANES,), jnp.float32)

        def start(c):
            return pltpu.async_copy(
                edge_hbm.at[pl.ds(N_EDGES + e_base + c * CHUNK, CHUNK)],
                bufs[c % 2], sems[c % 2])

        pending = start(0)
        for c in range(N_CHUNKS):
            nxt = start(c + 1) if c + 1 < N_CHUNKS else None
            pending.wait()
            buf = bufs[c % 2]

            @pl.loop(0, GROUPS)
            def _scatter(j):
                idx = buf[pl.ds(j * LANES, LANES)] + off
                sidx = lax.sort(idx)
                cnt, last = plsc.scan_count(sidx)
                plsc.addupdate_scatter(hist, [sidx], cnt.astype(jnp.float32),
                                       mask=last)

            pending = nxt

        pltpu.sync_copy(hist, out_hbm.at[wid])

    return hist_kernel(edge_flat, off16)


BN = 2048


def _tc_node_mlp(partial, W_e1, b_e1, W_e2, b_e2, W_n1, b_n1, W_n2, b_n2):
    grid = (pl.cdiv(N_NODES, BN),)

    dot = functools.partial(jnp.dot, preferred_element_type=jnp.float32)

    def body(part_ref, we1, be1, we2, be2, wn1, bn1, wn2, bn2, out_ref):
        h = jnp.maximum(we1[...] + be1[...][None, :], 0.0)
        v = dot(h, we2[...]) + be2[...][None, :]
        u = dot(v, wn1[...])
        deg = jnp.sum(part_ref[...], axis=0)
        t = deg[:, None] * u + bn1[...][None, :]
        z = jnp.maximum(t, 0.0)
        out_ref[...] = dot(z, wn2[...]) + bn2[...][None, :]

    full = lambda shape: pl.BlockSpec(shape, lambda i: tuple(0 for _ in shape))
    return pl.pallas_call(
        body,
        grid=grid,
        in_specs=[
            pl.BlockSpec((NW, BN), lambda i: (0, i)),
            full((1, NDIM)), full((NDIM,)),
            full((NDIM, NDIM)), full((NDIM,)),
            full((NDIM, NDIM)), full((NDIM,)),
            full((NDIM, NDIM)), full((NDIM,)),
        ],
        out_specs=pl.BlockSpec((BN, NDIM), lambda i: (i, 0)),
        out_shape=jax.ShapeDtypeStruct((N_NODES, NDIM), jnp.float32),
    )(partial, W_e1, b_e1, W_e2, b_e2, W_n1, b_n1, W_n2, b_n2)


def kernel(edge_index, edge_weight, num_nodes, W_e1, b_e1, W_e2, b_e2,
           W_n1, b_n1, W_n2, b_n2):
    del edge_weight
    off = jnp.asarray(num_nodes, jnp.int32) - jnp.int32(N_NODES)
    off16 = jnp.full((LANES,), off, jnp.int32)
    partial = _sc_histogram(edge_index.reshape(-1), off16)
    return _tc_node_mlp(partial, W_e1, b_e1, W_e2, b_e2, W_n1, b_n1, W_n2, b_n2)


# speedup vs baseline: 27.3099x; 1.1449x over previous
import functools

import jax
import jax.numpy as jnp
from jax import lax
from jax.experimental import pallas as pl
from jax.experimental.pallas import tpu as pltpu
from jax.experimental.pallas import tpu_sc as plsc

N_NODES = 100000
N_EDGES = 1600000
NDIM = 32

NUM_CORES = 2
NUM_SUBCORES = 16
NW = NUM_CORES * NUM_SUBCORES
E_PER_W = N_EDGES // NW
CHUNK = 10000
N_CHUNKS = E_PER_W // CHUNK
GROUPS = CHUNK // 16
LANES = 16


def _sc_histogram(edge_flat, off16):
    mesh = plsc.VectorSubcoreMesh(
        core_axis_name="c", subcore_axis_name="s",
        num_cores=NUM_CORES, num_subcores=NUM_SUBCORES)

    @functools.partial(
        pl.kernel,
        out_type=jax.ShapeDtypeStruct((NW, N_NODES), jnp.float32),
        mesh=mesh,
        scratch_types=[
            pltpu.VMEM((N_NODES,), jnp.float32),
            pltpu.VMEM((CHUNK,), jnp.int32),
            pltpu.VMEM((CHUNK,), jnp.int32),
            pltpu.VMEM((LANES,), jnp.int32),
            pltpu.SemaphoreType.DMA,
            pltpu.SemaphoreType.DMA,
        ],
        compiler_params=pltpu.CompilerParams(needs_layout_passes=False),
    )
    def hist_kernel(edge_hbm, off_hbm, out_hbm, hist, idxbuf0, idxbuf1,
                    off_v, sem0, sem1):
        wid = lax.axis_index("s") * NUM_CORES + lax.axis_index("c")
        e_base = wid * E_PER_W
        sems = (sem0, sem1)
        bufs = (idxbuf0, idxbuf1)

        pltpu.sync_copy(off_hbm, off_v)
        off = off_v[...]

        @pl.loop(0, N_NODES // LANES, unroll=8)
        def _zero(i):
            hist[pl.ds(i * LANES, LANES)] = jnp.zeros((LANES,), jnp.float32)

        ones = jnp.ones((LANES,), jnp.float32)

        def start(c):
            return pltpu.async_copy(
                edge_hbm.at[pl.ds(N_EDGES + e_base + c * CHUNK, CHUNK)],
                bufs[c % 2], sems[c % 2])

        pending = start(0)
        for c in range(N_CHUNKS):
            nxt = start(c + 1) if c + 1 < N_CHUNKS else None
            pending.wait()
            buf = bufs[c % 2]

            @pl.loop(0, GROUPS, unroll=4)
            def _scatter(j):
                idx = buf[pl.ds(j * LANES, LANES)] + off
                cnt, last = plsc.scan_count(idx)
                plsc.addupdate_scatter(hist, [idx], cnt.astype(jnp.float32),
                                       mask=last)

            pending = nxt

        pltpu.sync_copy(hist, out_hbm.at[wid])

    return hist_kernel(edge_flat, off16)


BN = 2048


def _tc_node_mlp(partial, W_e1, b_e1, W_e2, b_e2, W_n1, b_n1, W_n2, b_n2):
    grid = (pl.cdiv(N_NODES, BN),)

    dot = functools.partial(jnp.dot, preferred_element_type=jnp.float32)

    def body(part_ref, we1, be1, we2, be2, wn1, bn1, wn2, bn2, out_ref):
        h = jnp.maximum(we1[...] + be1[...][None, :], 0.0)
        v = dot(h, we2[...]) + be2[...][None, :]
        u = dot(v, wn1[...])
        deg = jnp.sum(part_ref[...], axis=0)
        t = deg[:, None] * u + bn1[...][None, :]
        z = jnp.maximum(t, 0.0)
        out_ref[...] = dot(z, wn2[...]) + bn2[...][None, :]

    full = lambda shape: pl.BlockSpec(shape, lambda i: tuple(0 for _ in shape))
    return pl.pallas_call(
        body,
        grid=grid,
        in_specs=[
            pl.BlockSpec((NW, BN), lambda i: (0, i)),
            full((1, NDIM)), full((NDIM,)),
            full((NDIM, NDIM)), full((NDIM,)),
            full((NDIM, NDIM)), full((NDIM,)),
            full((NDIM, NDIM)), full((NDIM,)),
        ],
        out_specs=pl.BlockSpec((BN, NDIM), lambda i: (i, 0)),
        out_shape=jax.ShapeDtypeStruct((N_NODES, NDIM), jnp.float32),
    )(partial, W_e1, b_e1, W_e2, b_e2, W_n1, b_n1, W_n2, b_n2)


def kernel(edge_index, edge_weight, num_nodes, W_e1, b_e1, W_e2, b_e2,
           W_n1, b_n1, W_n2, b_n2):
    del edge_weight
    off = jnp.asarray(num_nodes, jnp.int32) - jnp.int32(N_NODES)
    off16 = jnp.full((LANES,), off, jnp.int32)
    partial = _sc_histogram(edge_index.reshape(-1), off16)
    return _tc_node_mlp(partial, W_e1, b_e1, W_e2, b_e2, W_n1, b_n1, W_n2, b_n2)


# speedup vs baseline: 30.9737x; 1.1342x over previous
import functools

import jax
import jax.numpy as jnp
from jax import lax
from jax.experimental import pallas as pl
from jax.experimental.pallas import tpu as pltpu
from jax.experimental.pallas import tpu_sc as plsc

N_NODES = 100000
N_EDGES = 1600000
NDIM = 32

NUM_CORES = 2
NUM_SUBCORES = 16
NW = NUM_CORES * NUM_SUBCORES
E_PER_W = N_EDGES // NW
CHUNK = 10000
N_CHUNKS = E_PER_W // CHUNK
GROUPS = CHUNK // 16
LANES = 16


def _sc_histogram(edge_flat, off16):
    mesh = plsc.VectorSubcoreMesh(
        core_axis_name="c", subcore_axis_name="s",
        num_cores=NUM_CORES, num_subcores=NUM_SUBCORES)

    @functools.partial(
        pl.kernel,
        out_type=jax.ShapeDtypeStruct((NW, N_NODES), jnp.float32),
        mesh=mesh,
        scratch_types=[
            pltpu.VMEM((N_NODES,), jnp.float32),
            pltpu.VMEM((CHUNK,), jnp.int32),
            pltpu.VMEM((CHUNK,), jnp.int32),
            pltpu.VMEM((LANES,), jnp.int32),
            pltpu.SemaphoreType.DMA,
            pltpu.SemaphoreType.DMA,
        ],
        compiler_params=pltpu.CompilerParams(needs_layout_passes=False),
    )
    def hist_kernel(edge_hbm, off_hbm, out_hbm, hist, idxbuf0, idxbuf1,
                    off_v, sem0, sem1):
        wid = lax.axis_index("s") * NUM_CORES + lax.axis_index("c")
        e_base = wid * E_PER_W
        sems = (sem0, sem1)
        bufs = (idxbuf0, idxbuf1)

        pltpu.sync_copy(off_hbm, off_v)
        off = off_v[...]

        @pl.loop(0, N_NODES // LANES, unroll=8)
        def _zero(i):
            hist[pl.ds(i * LANES, LANES)] = jnp.zeros((LANES,), jnp.float32)

        ones = jnp.ones((LANES,), jnp.float32)

        def start(c):
            return pltpu.async_copy(
                edge_hbm.at[pl.ds(N_EDGES + e_base + c * CHUNK, CHUNK)],
                bufs[c % 2], sems[c % 2])

        pending = start(0)
        for c in range(N_CHUNKS):
            nxt = start(c + 1) if c + 1 < N_CHUNKS else None
            pending.wait()
            buf = bufs[c % 2]

            @pl.loop(0, GROUPS, unroll=4)
            def _scatter(j):
                idx = buf[pl.ds(j * LANES, LANES)] + off
                cnt, last = plsc.scan_count(idx)
                plsc.addupdate_scatter(hist, [idx], cnt.astype(jnp.float32),
                                       mask=last)

            pending = nxt

        pltpu.sync_copy(hist, out_hbm.at[wid])

    return hist_kernel(edge_flat, off16)


BN = 8192


def _tc_node_mlp(partial, W_e1, b_e1, W_e2, b_e2, W_n1, b_n1, W_n2, b_n2):
    grid = (pl.cdiv(N_NODES, BN),)

    def body(part_ref, we1, be1, we2, be2, wn1, bn1, wn2, bn2, out_ref):
        h = jnp.maximum(we1[...] + be1[...][None, :], 0.0)
        v = jnp.sum(h.reshape(NDIM, 1) * we2[...], axis=0)[None, :]
        v = v + be2[...][None, :]
        u = jnp.sum(v.reshape(NDIM, 1) * wn1[...], axis=0)[None, :]
        deg = jnp.sum(part_ref[...], axis=0)
        t = deg[:, None] * u + bn1[...][None, :]
        z = jnp.maximum(t, 0.0)
        o = jnp.dot(z, wn2[...], preferred_element_type=jnp.float32)
        out_ref[...] = o + bn2[...][None, :]

    full = lambda shape: pl.BlockSpec(shape, lambda i: tuple(0 for _ in shape))
    return pl.pallas_call(
        body,
        grid=grid,
        in_specs=[
            pl.BlockSpec((NW, BN), lambda i: (0, i)),
            full((1, NDIM)), full((NDIM,)),
            full((NDIM, NDIM)), full((NDIM,)),
            full((NDIM, NDIM)), full((NDIM,)),
            full((NDIM, NDIM)), full((NDIM,)),
        ],
        out_specs=pl.BlockSpec((BN, NDIM), lambda i: (i, 0)),
        out_shape=jax.ShapeDtypeStruct((N_NODES, NDIM), jnp.float32),
    )(partial, W_e1, b_e1, W_e2, b_e2, W_n1, b_n1, W_n2, b_n2)


def kernel(edge_index, edge_weight, num_nodes, W_e1, b_e1, W_e2, b_e2,
           W_n1, b_n1, W_n2, b_n2):
    del edge_weight
    off = jnp.asarray(num_nodes, jnp.int32) - jnp.int32(N_NODES)
    off16 = jnp.full((LANES,), off, jnp.int32)
    partial = _sc_histogram(edge_index.reshape(-1), off16)
    return _tc_node_mlp(partial, W_e1, b_e1, W_e2, b_e2, W_n1, b_n1, W_n2, b_n2)


# speedup vs baseline: 42.0634x; 1.3580x over previous
import functools

import jax
import jax.numpy as jnp
from jax import lax
from jax.experimental import pallas as pl
from jax.experimental.pallas import tpu as pltpu
from jax.experimental.pallas import tpu_sc as plsc

N_NODES = 100000
N_EDGES = 1600000
NDIM = 32

NUM_CORES = 2
NUM_SUBCORES = 16
NW = NUM_CORES * NUM_SUBCORES
E_PER_W = N_EDGES // NW
CHUNK = 10000
N_CHUNKS = E_PER_W // CHUNK
GROUPS = CHUNK // 16
LANES = 16


def _sc_histogram(edge_flat, off16):
    mesh = plsc.VectorSubcoreMesh(
        core_axis_name="c", subcore_axis_name="s",
        num_cores=NUM_CORES, num_subcores=NUM_SUBCORES)

    @functools.partial(
        pl.kernel,
        out_type=jax.ShapeDtypeStruct((NW, N_NODES), jnp.float32),
        mesh=mesh,
        scratch_types=[
            pltpu.VMEM((N_NODES,), jnp.float32),
            pltpu.VMEM((CHUNK,), jnp.int32),
            pltpu.VMEM((CHUNK,), jnp.int32),
            pltpu.VMEM((LANES,), jnp.int32),
            pltpu.SemaphoreType.DMA,
            pltpu.SemaphoreType.DMA,
        ],
        compiler_params=pltpu.CompilerParams(needs_layout_passes=False),
    )
    def hist_kernel(edge_hbm, off_hbm, out_hbm, hist, idxbuf0, idxbuf1,
                    off_v, sem0, sem1):
        wid = lax.axis_index("s") * NUM_CORES + lax.axis_index("c")
        e_base = wid * E_PER_W
        sems = (sem0, sem1)
        bufs = (idxbuf0, idxbuf1)

        pltpu.sync_copy(off_hbm, off_v)
        off = off_v[...]

        @plsc.parallel_loop(0, N_NODES // LANES, unroll=8)
        def _zero(i):
            hist[pl.ds(i * LANES, LANES)] = jnp.zeros((LANES,), jnp.float32)

        ones = jnp.ones((LANES,), jnp.float32)

        def start(c):
            return pltpu.async_copy(
                edge_hbm.at[pl.ds(N_EDGES + e_base + c * CHUNK, CHUNK)],
                bufs[c % 2], sems[c % 2])

        pending = start(0)
        for c in range(N_CHUNKS):
            nxt = start(c + 1) if c + 1 < N_CHUNKS else None
            pending.wait()
            buf = bufs[c % 2]

            @plsc.parallel_loop(0, GROUPS, unroll=8)
            def _scatter(j):
                idx = buf[pl.ds(j * LANES, LANES)] + off
                cnt, last = plsc.scan_count(idx)
                plsc.addupdate_scatter(hist, [idx], cnt.astype(jnp.float32),
                                       mask=last)

            pending = nxt

        pltpu.sync_copy(hist, out_hbm.at[wid])

    return hist_kernel(edge_flat, off16)


BN = 8192


def _tc_node_mlp(partial, W_e1, b_e1, W_e2, b_e2, W_n1, b_n1, W_n2, b_n2):
    grid = (pl.cdiv(N_NODES, BN),)

    def body(part_ref, we1, be1, we2, be2, wn1, bn1, wn2, bn2, out_ref):
        h = jnp.maximum(we1[...] + be1[...][None, :], 0.0)
        v = jnp.sum(h.reshape(NDIM, 1) * we2[...], axis=0)[None, :]
        v = v + be2[...][None, :]
        u = jnp.sum(v.reshape(NDIM, 1) * wn1[...], axis=0)[None, :]
        deg = jnp.sum(part_ref[...], axis=0)
        t = deg[:, None] * u + bn1[...][None, :]
        z = jnp.maximum(t, 0.0)
        o = jnp.dot(z, wn2[...], preferred_element_type=jnp.float32)
        out_ref[...] = o + bn2[...][None, :]

    full = lambda shape: pl.BlockSpec(shape, lambda i: tuple(0 for _ in shape))
    return pl.pallas_call(
        body,
        grid=grid,
        in_specs=[
            pl.BlockSpec((NW, BN), lambda i: (0, i)),
            full((1, NDIM)), full((NDIM,)),
            full((NDIM, NDIM)), full((NDIM,)),
            full((NDIM, NDIM)), full((NDIM,)),
            full((NDIM, NDIM)), full((NDIM,)),
        ],
        out_specs=pl.BlockSpec((BN, NDIM), lambda i: (i, 0)),
        out_shape=jax.ShapeDtypeStruct((N_NODES, NDIM), jnp.float32),
    )(partial, W_e1, b_e1, W_e2, b_e2, W_n1, b_n1, W_n2, b_n2)


def kernel(edge_index, edge_weight, num_nodes, W_e1, b_e1, W_e2, b_e2,
           W_n1, b_n1, W_n2, b_n2):
    del edge_weight
    off = jnp.asarray(num_nodes, jnp.int32) - jnp.int32(N_NODES)
    off16 = jnp.full((LANES,), off, jnp.int32)
    partial = _sc_histogram(edge_index.reshape(-1), off16)
    return _tc_node_mlp(partial, W_e1, b_e1, W_e2, b_e2, W_n1, b_n1, W_n2, b_n2)


# speedup vs baseline: 46.4756x; 1.1049x over previous
import functools

import jax
import jax.numpy as jnp
from jax import lax
from jax.experimental import pallas as pl
from jax.experimental.pallas import tpu as pltpu
from jax.experimental.pallas import tpu_sc as plsc

N_NODES = 100000
N_EDGES = 1600000
NDIM = 32

NUM_CORES = 2
NUM_SUBCORES = 16
NW = NUM_CORES * NUM_SUBCORES
CHUNK = 3200
N_CHUNKS_TOT = N_EDGES // CHUNK
ITERS = -(-N_CHUNKS_TOT // NW)
GROUPS = CHUNK // 16
LANES = 16


def _sc_histogram(edge_index, off16):
    mesh = plsc.VectorSubcoreMesh(
        core_axis_name="c", subcore_axis_name="s",
        num_cores=NUM_CORES, num_subcores=NUM_SUBCORES)

    @functools.partial(
        pl.kernel,
        out_type=jax.ShapeDtypeStruct((NW, N_NODES), jnp.float32),
        mesh=mesh,
        scratch_types=[
            pltpu.VMEM((N_NODES,), jnp.float32),
            pltpu.VMEM((2, CHUNK), jnp.int32),
            pltpu.VMEM((LANES,), jnp.int32),
            pltpu.SemaphoreType.DMA,
        ],
        compiler_params=pltpu.CompilerParams(needs_layout_passes=False),
    )
    def hist_kernel(edge_hbm, off_hbm, out_hbm, hist, buf, off_v, sem):
        wid = lax.axis_index("s") * NUM_CORES + lax.axis_index("c")

        pltpu.sync_copy(off_hbm, off_v)
        off = off_v[...]

        @plsc.parallel_loop(0, N_NODES // LANES, unroll=8)
        def _zero(i):
            hist[pl.ds(i * LANES, LANES)] = jnp.zeros((LANES,), jnp.float32)

        for i in range(ITERS):
            cid = i * NW + wid

            @pl.when(cid < N_CHUNKS_TOT)
            def _chunk():
                base = pl.multiple_of(cid * CHUNK, 128)
                pltpu.async_copy(
                    edge_hbm.at[:, pl.ds(base, CHUNK)], buf, sem).wait()

                @plsc.parallel_loop(0, GROUPS, unroll=8)
                def _scatter(j):
                    idx = buf[1, pl.ds(j * LANES, LANES)] + off
                    cnt, last = plsc.scan_count(idx)
                    plsc.addupdate_scatter(hist, [idx],
                                           cnt.astype(jnp.float32),
                                           mask=last)

        pltpu.sync_copy(hist, out_hbm.at[wid])

    return hist_kernel(edge_index, off16)


BN = 8192


def _tc_node_mlp(partial, W_e1, b_e1, W_e2, b_e2, W_n1, b_n1, W_n2, b_n2):
    grid = (pl.cdiv(N_NODES, BN),)

    def body(part_ref, we1, be1, we2, be2, wn1, bn1, wn2, bn2, out_ref):
        h = jnp.maximum(we1[...] + be1[...][None, :], 0.0)
        v = jnp.sum(h.reshape(NDIM, 1) * we2[...], axis=0)[None, :]
        v = v + be2[...][None, :]
        u = jnp.sum(v.reshape(NDIM, 1) * wn1[...], axis=0)[None, :]
        deg = jnp.sum(part_ref[...], axis=0)
        t = deg[:, None] * u + bn1[...][None, :]
        z = jnp.maximum(t, 0.0)
        o = jnp.dot(z, wn2[...], preferred_element_type=jnp.float32)
        out_ref[...] = o + bn2[...][None, :]

    full = lambda shape: pl.BlockSpec(shape, lambda i: tuple(0 for _ in shape))
    return pl.pallas_call(
        body,
        grid=grid,
        in_specs=[
            pl.BlockSpec((NW, BN), lambda i: (0, i)),
            full((1, NDIM)), full((NDIM,)),
            full((NDIM, NDIM)), full((NDIM,)),
            full((NDIM, NDIM)), full((NDIM,)),
            full((NDIM, NDIM)), full((NDIM,)),
        ],
        out_specs=pl.BlockSpec((BN, NDIM), lambda i: (i, 0)),
        out_shape=jax.ShapeDtypeStruct((N_NODES, NDIM), jnp.float32),
    )(partial, W_e1, b_e1, W_e2, b_e2, W_n1, b_n1, W_n2, b_n2)


def kernel(edge_index, edge_weight, num_nodes, W_e1, b_e1, W_e2, b_e2,
           W_n1, b_n1, W_n2, b_n2):
    del edge_weight
    off = jnp.asarray(num_nodes, jnp.int32) - jnp.int32(N_NODES)
    off16 = jnp.full((LANES,), off, jnp.int32)
    partial = _sc_histogram(edge_index, off16)
    return _tc_node_mlp(partial, W_e1, b_e1, W_e2, b_e2, W_n1, b_n1, W_n2, b_n2)
